# baseline (speedup 1.0000x reference)
import functools

import jax
import jax.numpy as jnp
from jax import lax
from jax.experimental import pallas as pl
from jax.experimental.pallas import tpu as pltpu
from jax.experimental.pallas import tpu_sc as plsc

NC = 2
NS = 16
NW = NC * NS
L = 16

B = 16384
D = 16
V = 1000000
BPW = B // NW
IDXC = 128
KCH = BPW // IDXC

CPT = 2048
NFULL = V // CPT
CPS = (NFULL + NS - 1) // NS
TAIL0 = NFULL * CPT
TAILW = 512
PATCH0 = TAIL0 + TAILW
PATCHW = V - PATCH0


def _transpose_chunk(vblk, tflat, width):
    iota = lax.iota(jnp.int32, L)

    def group(g, carry):
        for j in range(L):
            r = g * L + j
            rsplat = jnp.full((L,), 0, jnp.int32) + r
            col = plsc.load_gather(vblk, [iota, rsplat])
            plsc.store_scatter(tflat, [rsplat * D + iota], col)
        return carry

    lax.fori_loop(0, width // L, group, 0)


def _transpose_body(table_r, tail_r, out_r, vblk, tflat, sem, s):
    def chunk(j, carry):
        ci = jnp.minimum(s + NS * j, NFULL - 1)
        c0 = ci * CPT
        pltpu.async_copy(table_r.at[:, pl.ds(c0, CPT)], vblk, sem).wait()
        _transpose_chunk(vblk, tflat, CPT)
        pltpu.async_copy(tflat, out_r.at[pl.ds(c0 * D, CPT * D)], sem).wait()
        return carry

    lax.fori_loop(0, CPS, chunk, 0)

    @pl.when(s == 8)
    def _tail():
        pltpu.async_copy(table_r.at[:, pl.ds(TAIL0, TAILW)],
                         vblk.at[:, pl.ds(0, TAILW)], sem).wait()
        _transpose_chunk(vblk, tflat, TAILW)
        pltpu.async_copy(tflat.at[pl.ds(0, TAILW * D)],
                         out_r.at[pl.ds(TAIL0 * D, TAILW * D)], sem).wait()

    @pl.when(s == 9)
    def _patch():
        pltpu.async_copy(tail_r, tflat.at[pl.ds(0, PATCHW * D)], sem).wait()
        pltpu.async_copy(tflat.at[pl.ds(0, PATCHW * D)],
                         out_r.at[pl.ds(PATCH0 * D, PATCHW * D)], sem).wait()


@functools.partial(
    pl.kernel,
    out_type=(jax.ShapeDtypeStruct((V * D,), jnp.float32),
              jax.ShapeDtypeStruct((V * D,), jnp.float32)),
    mesh=plsc.VectorSubcoreMesh(core_axis_name="c", subcore_axis_name="s"),
    compiler_params=pltpu.CompilerParams(needs_layout_passes=False),
    scratch_types=[
        pltpu.VMEM((D, CPT), jnp.float32),
        pltpu.VMEM((CPT * D,), jnp.float32),
        pltpu.SemaphoreType.DMA,
    ],
)
def _sc_transpose(ut_r, it_r, utail_r, itail_r, ulin_r, ilin_r,
                  vblk, tflat, sem):
    c = lax.axis_index("c")
    s = lax.axis_index("s")

    @pl.when(c == 0)
    def _users():
        _transpose_body(ut_r, utail_r, ulin_r, vblk, tflat, sem, s)

    @pl.when(c == 1)
    def _items():
        _transpose_body(it_r, itail_r, ilin_r, vblk, tflat, sem, s)


def _gather_body(users_r, items_r, ut_r, it_r, out_r,
                 uidx, iidx, urows, irows, outv, sem):
    w = lax.axis_index("s") * NC + lax.axis_index("c")
    base = w * BPW

    pltpu.sync_copy(users_r.at[pl.ds(base, BPW)], uidx)
    pltpu.sync_copy(items_r.at[pl.ds(base, BPW)], iidx)

    handles = []
    for k in range(KCH):
        sl = pl.ds(k * IDXC, IDXC)
        handles.append(pltpu.async_copy(
            ut_r.at[uidx.at[sl]], urows.at[pl.ds(k * IDXC, IDXC), :], sem))
        handles.append(pltpu.async_copy(
            it_r.at[iidx.at[sl]], irows.at[pl.ds(k * IDXC, IDXC), :], sem))
    for h in handles:
        h.wait()

    iota = lax.iota(jnp.int32, L)

    def group(g, carry):
        row_idx = iota + g * L
        acc = jnp.zeros((L,), jnp.float32)
        for d in range(D):
            dcol = jnp.full((L,), d, jnp.int32)
            u = plsc.load_gather(urows, [row_idx, dcol])
            v = plsc.load_gather(irows, [row_idx, dcol])
            acc = acc + u * v
        outv[pl.ds(g * L, L)] = acc
        return carry

    lax.fori_loop(0, BPW // L, group, 0)

    pltpu.sync_copy(outv, out_r.at[pl.ds(base, BPW)])


@functools.partial(
    pl.kernel,
    out_type=jax.ShapeDtypeStruct((B,), jnp.float32),
    mesh=plsc.VectorSubcoreMesh(core_axis_name="c", subcore_axis_name="s"),
    compiler_params=pltpu.CompilerParams(
        needs_layout_passes=False, use_tc_tiling_on_sc=False),
    scratch_types=[
        pltpu.VMEM((BPW,), jnp.int32),
        pltpu.VMEM((BPW,), jnp.int32),
        pltpu.VMEM((BPW, D), jnp.float32),
        pltpu.VMEM((BPW, D), jnp.float32),
        pltpu.VMEM((BPW,), jnp.float32),
        pltpu.SemaphoreType.DMA,
    ],
)
def _sc_gather_dot(users_r, items_r, ut_r, it_r, out_r,
                   uidx, iidx, urows, irows, outv, sem):
    _gather_body(users_r, items_r, ut_r, it_r, out_r,
                 uidx, iidx, urows, irows, outv, sem)


def kernel(users, items, users_table, items_table):
    ulin, ilin = _sc_transpose(users_table.T, items_table.T,
                               users_table[PATCH0:, :].reshape(-1),
                               items_table[PATCH0:, :].reshape(-1))
    out = _sc_gather_dot(users.astype(jnp.int32), items.astype(jnp.int32),
                         ulin.reshape(V, D), ilin.reshape(V, D))
    return out.reshape(B, 1)


# speedup vs baseline: 1.1196x; 1.1196x over previous
import functools

import jax
import jax.numpy as jnp
from jax import lax
from jax.experimental import pallas as pl
from jax.experimental.pallas import tpu as pltpu
from jax.experimental.pallas import tpu_sc as plsc

NC = 2
NS = 16
NW = NC * NS
L = 16

B = 16384
D = 16
BPW = B // NW
IDXC = 128
KCH = BPW // IDXC


def _body(users_r, items_r, ut_r, it_r, out_r,
          uidx, iidx, urows, irows, outv, sem):
    w = lax.axis_index("s") * NC + lax.axis_index("c")
    base = w * BPW

    pltpu.sync_copy(users_r.at[pl.ds(base, BPW)], uidx)
    pltpu.sync_copy(items_r.at[pl.ds(base, BPW)], iidx)

    handles = []
    for k in range(KCH):
        sl = pl.ds(k * IDXC, IDXC)
        handles.append(pltpu.async_copy(
            ut_r.at[uidx.at[sl]], urows.at[pl.ds(k * IDXC, IDXC), :], sem))
        handles.append(pltpu.async_copy(
            it_r.at[iidx.at[sl]], irows.at[pl.ds(k * IDXC, IDXC), :], sem))
    for h in handles:
        h.wait()

    iota = lax.iota(jnp.int32, L)

    def group(g, carry):
        row_idx = iota + g * L
        acc = jnp.zeros((L,), jnp.float32)
        for d in range(D):
            dcol = jnp.full((L,), d, jnp.int32)
            u = plsc.load_gather(urows, [row_idx, dcol])
            v = plsc.load_gather(irows, [row_idx, dcol])
            acc = acc + u * v
        outv[pl.ds(g * L, L)] = acc
        return carry

    lax.fori_loop(0, BPW // L, group, 0)

    pltpu.sync_copy(outv, out_r.at[pl.ds(base, BPW)])


@functools.partial(
    pl.kernel,
    out_type=jax.ShapeDtypeStruct((B,), jnp.float32),
    mesh=plsc.VectorSubcoreMesh(core_axis_name="c", subcore_axis_name="s"),
    compiler_params=pltpu.CompilerParams(
        needs_layout_passes=False, use_tc_tiling_on_sc=False),
    scratch_types=[
        pltpu.VMEM((BPW,), jnp.int32),
        pltpu.VMEM((BPW,), jnp.int32),
        pltpu.VMEM((BPW, D), jnp.float32),
        pltpu.VMEM((BPW, D), jnp.float32),
        pltpu.VMEM((BPW,), jnp.float32),
        pltpu.SemaphoreType.DMA,
    ],
)
def _sc_kernel(users_r, items_r, ut_r, it_r, out_r,
               uidx, iidx, urows, irows, outv, sem):
    _body(users_r, items_r, ut_r, it_r, out_r,
          uidx, iidx, urows, irows, outv, sem)


def kernel(users, items, users_table, items_table):
    out = _sc_kernel(users.astype(jnp.int32), items.astype(jnp.int32),
                     users_table, items_table)
    return out.reshape(B, 1)
